# baseline (speedup 1.0000x reference)
import jax
import jax.numpy as jnp
from jax.experimental import pallas as pl
from jax.experimental.pallas import tpu as pltpu

N, B, D, LD, C = 8192, 128, 128, 32, 64
BLK = 2048
NB = N // BLK
CD = D * 5
CLD = LD * 4
HI = jax.lax.Precision.HIGHEST


def _chunks7(v):
    parts = [((v >> s) & 127).astype(jnp.bfloat16) for s in (0, 7, 14, 21, 28)]
    return jnp.concatenate(parts, axis=-1)


def _chunks8(v):
    parts = [((v >> s) & 255).astype(jnp.bfloat16) for s in (0, 8, 16, 24)]
    return jnp.concatenate(parts, axis=-1)


def _bits(f):
    return jax.lax.bitcast_convert_type(f, jnp.int32)


def _dot_t(a, b, prec=None):
    return jax.lax.dot_general(a, b, (((1,), (1,)), ((), ())),
                               precision=prec, preferred_element_type=jnp.float32)


def _dot(a, b, prec=None):
    return jax.lax.dot_general(a, b, (((1,), (0,)), ((), ())),
                               precision=prec, preferred_element_type=jnp.float32)


def _match_kernel(x_ref, sf_ref, midx_ref, xc_ref):
    j = pl.program_id(0)

    @pl.when(j == 0)
    def _init():
        xc_ref[...] = _chunks7(_bits(x_ref[...]))
        midx_ref[...] = jnp.full_like(midx_ref, N)

    sfc = _chunks7(_bits(sf_ref[...]))
    xc = xc_ref[...]
    g = _dot_t(xc, sfc)
    nx = jnp.sum(xc.astype(jnp.float32) ** 2, axis=1)
    nf = jnp.sum(sfc.astype(jnp.float32) ** 2, axis=1)
    m2 = nx[:, None] + nf[None, :] - 2.0 * g
    il = jax.lax.broadcasted_iota(jnp.int32, (B, BLK), 1)
    lidx = jnp.min(jnp.where(m2 == 0.0, il, BLK), axis=1)
    cand = jnp.where(lidx < BLK, j * BLK + lidx, N)
    midx_ref[0, :] = jnp.minimum(midx_ref[0, :], cand)


def _gather_heads_kernel(midx_ref, d1f_ref, d2f_ref, w1_ref, b1_ref,
                         w2_ref, b2_ref, u1_ref, u2_ref,
                         x1_ref, x2_ref, oh1_ref, oh2_ref,
                         nx1_ref, nx2_ref, u1c_ref, u2c_ref):
    ohq = (midx_ref[0, :][:, None]
           == jax.lax.broadcasted_iota(jnp.int32, (B, N), 1)
           ).astype(jnp.bfloat16)
    sides = (
        (d1f_ref, w1_ref, b1_ref, u1_ref, x1_ref, oh1_ref, nx1_ref, u1c_ref),
        (d2f_ref, w2_ref, b2_ref, u2_ref, x2_ref, oh2_ref, nx2_ref, u2c_ref),
    )
    for (df_ref, w_ref, b_ref, u_ref, x_ref, oh_ref, nx_ref, uc_ref) in sides:
        df = df_ref[...]
        hi = df.astype(jnp.bfloat16)
        mid = (df - hi.astype(jnp.float32)).astype(jnp.bfloat16)
        xg = _dot(ohq, hi) + _dot(ohq, mid)
        x_ref[...] = xg
        nx_ref[0, :] = jnp.sum(xg * xg, axis=1)
        u = u_ref[...]
        uc_ref[...] = _chunks8(_bits(u))
        y = _dot(xg, w_ref[...], HI) + b_ref[0, :][None, :]
        ny = jnp.sum(y * y, axis=1)
        nuf = jnp.sum(u * u, axis=1)
        dq = ny[:, None] + nuf[None, :] - 2.0 * _dot_t(y, u, HI)
        mn = jnp.min(dq, axis=1, keepdims=True)
        cb = jax.lax.broadcasted_iota(jnp.int32, (B, C), 1)
        cidx = jnp.min(jnp.where(dq == mn, cb, C), axis=1)
        oh_ref[...] = (cb == cidx[:, None]).astype(jnp.bfloat16)


def _agg_kernel(x1_ref, x2_ref, oh1_ref, oh2_ref, nx1_ref, nx2_ref,
                u1c_ref, u2c_ref,
                d1f_ref, d1l_ref, d2f_ref, d2l_ref, slb_ref, out_ref,
                num1_ref, num2_ref):
    j = pl.program_id(0)

    @pl.when(j == 0)
    def _init():
        num1_ref[...] = jnp.zeros_like(num1_ref)
        num2_ref[...] = jnp.zeros_like(num2_ref)

    slb = slb_ref[...]
    slb_ext = jnp.concatenate(
        [slb, jnp.ones((BLK, 1), jnp.float32)], axis=1).astype(jnp.bfloat16)
    sides = (
        (x1_ref, oh1_ref, nx1_ref, u1c_ref, d1f_ref, d1l_ref, num1_ref),
        (x2_ref, oh2_ref, nx2_ref, u2c_ref, d2f_ref, d2l_ref, num2_ref),
    )
    for (x_ref, oh_ref, nx_ref, uc_ref, df_ref, dl_ref, num_ref) in sides:
        f = df_ref[...]
        uc = uc_ref[...]
        lc = _chunks8(_bits(dl_ref[...]))
        nl = jnp.sum(lc.astype(jnp.float32) ** 2, axis=1)
        nu = jnp.sum(uc.astype(jnp.float32) ** 2, axis=1)
        m2l = nl[:, None] + nu[None, :] - 2.0 * _dot_t(lc, uc)
        e = (m2l == 0.0).astype(jnp.bfloat16)
        mask = _dot_t(oh_ref[...], e)
        g = _dot_t(x_ref[...].astype(jnp.bfloat16),
                   f.astype(jnp.bfloat16))
        nf = jnp.sum(f * f, axis=1)
        sq = nx_ref[0, :][:, None] + nf[None, :] - 2.0 * g
        expo = (jnp.exp(-sq) * mask).astype(jnp.bfloat16)
        num_ref[...] += _dot(expo, slb_ext)

    @pl.when(j == NB - 1)
    def _fin():
        n1 = num1_ref[...]
        n2 = num2_ref[...]
        out_ref[...] = 0.5 * (n1[:, :LD] / n1[:, LD:LD + 1]
                              + n2[:, :LD] / n2[:, LD:LD + 1])


def kernel(x, star_features, star_labels, d1_features, d1_labels,
           d2_features, d2_labels, unique1, unique2, W1, b1, W2, b2):
    f32 = jnp.float32
    bf16 = jnp.bfloat16
    midx = pl.pallas_call(
        _match_kernel,
        grid=(NB,),
        in_specs=[
            pl.BlockSpec((B, D), lambda j: (0, 0)),
            pl.BlockSpec((BLK, D), lambda j: (j, 0)),
        ],
        out_specs=pl.BlockSpec((1, B), lambda j: (0, 0)),
        out_shape=jax.ShapeDtypeStruct((1, B), jnp.int32),
        scratch_shapes=[
            pltpu.VMEM((B, CD), bf16),
        ],
    )(x, star_features)

    const2 = lambda j: (0, 0)
    x1, x2, oh1, oh2, nx1, nx2, u1c, u2c = pl.pallas_call(
        _gather_heads_kernel,
        in_specs=[pl.BlockSpec(s, None) for s in
                  ((1, B), (N, D), (N, D), (D, LD), (1, LD), (D, LD), (1, LD),
                   (C, LD), (C, LD))],
        out_shape=[
            jax.ShapeDtypeStruct((B, D), f32),
            jax.ShapeDtypeStruct((B, D), f32),
            jax.ShapeDtypeStruct((B, C), bf16),
            jax.ShapeDtypeStruct((B, C), bf16),
            jax.ShapeDtypeStruct((1, B), f32),
            jax.ShapeDtypeStruct((1, B), f32),
            jax.ShapeDtypeStruct((C, CLD), bf16),
            jax.ShapeDtypeStruct((C, CLD), bf16),
        ],
    )(midx, d1_features, d2_features, W1, b1.reshape(1, LD),
      W2, b2.reshape(1, LD), unique1, unique2)

    s = pl.pallas_call(
        _agg_kernel,
        grid=(NB,),
        in_specs=[
            pl.BlockSpec((B, D), const2),
            pl.BlockSpec((B, D), const2),
            pl.BlockSpec((B, C), const2),
            pl.BlockSpec((B, C), const2),
            pl.BlockSpec((1, B), const2),
            pl.BlockSpec((1, B), const2),
            pl.BlockSpec((C, CLD), const2),
            pl.BlockSpec((C, CLD), const2),
            pl.BlockSpec((BLK, D), lambda j: (j, 0)),
            pl.BlockSpec((BLK, LD), lambda j: (j, 0)),
            pl.BlockSpec((BLK, D), lambda j: (j, 0)),
            pl.BlockSpec((BLK, LD), lambda j: (j, 0)),
            pl.BlockSpec((BLK, LD), lambda j: (j, 0)),
        ],
        out_specs=pl.BlockSpec((B, LD), const2),
        out_shape=jax.ShapeDtypeStruct((B, LD), f32),
        scratch_shapes=[
            pltpu.VMEM((B, LD + 1), f32),
            pltpu.VMEM((B, LD + 1), f32),
        ],
    )(x1, x2, oh1, oh2, nx1, nx2, u1c, u2c,
      d1_features, d1_labels, d2_features, d2_labels, star_labels)
    return s


# speedup vs baseline: 1.0168x; 1.0168x over previous
import jax
import jax.numpy as jnp
from jax.experimental import pallas as pl
from jax.experimental.pallas import tpu as pltpu

N, B, D, LD, C = 8192, 128, 128, 32, 64
BLK = 2048
NB = N // BLK
CD = D * 5
CLD = LD * 4
HI = jax.lax.Precision.HIGHEST


def _chunks7(v):
    parts = [((v >> s) & 127).astype(jnp.bfloat16) for s in (0, 7, 14, 21, 28)]
    return jnp.concatenate(parts, axis=-1)


def _chunks8(v):
    parts = [((v >> s) & 255).astype(jnp.bfloat16) for s in (0, 8, 16, 24)]
    return jnp.concatenate(parts, axis=-1)


def _bits(f):
    return jax.lax.bitcast_convert_type(f, jnp.int32)


def _dot_t(a, b, prec=None):
    return jax.lax.dot_general(a, b, (((1,), (1,)), ((), ())),
                               precision=prec, preferred_element_type=jnp.float32)


def _dot(a, b, prec=None):
    return jax.lax.dot_general(a, b, (((1,), (0,)), ((), ())),
                               precision=prec, preferred_element_type=jnp.float32)


def _fused_kernel(x_ref, sf_ref, d1f_ref, d2f_ref, w1_ref, b1_ref,
                  w2_ref, b2_ref, u1_ref, u2_ref,
                  d1l_ref, d2l_ref, slb_ref, out_ref,
                  xc_ref, midx_ref, x1_ref, x2_ref, oh1_ref, oh2_ref,
                  nx1_ref, nx2_ref, u1c_ref, u2c_ref, num1_ref, num2_ref):
    j = pl.program_id(0)
    sides = (
        (d1f_ref, w1_ref, b1_ref, u1_ref, d1l_ref,
         x1_ref, oh1_ref, nx1_ref, u1c_ref, num1_ref),
        (d2f_ref, w2_ref, b2_ref, u2_ref, d2l_ref,
         x2_ref, oh2_ref, nx2_ref, u2c_ref, num2_ref),
    )

    @pl.when(j == 0)
    def _init():
        xc_ref[...] = _chunks7(_bits(x_ref[...]))
        midx_ref[...] = jnp.full_like(midx_ref, N)
        num1_ref[...] = jnp.zeros_like(num1_ref)
        num2_ref[...] = jnp.zeros_like(num2_ref)

    @pl.when(j < NB)
    def _match():
        sfc = _chunks7(_bits(sf_ref[...]))
        xc = xc_ref[...]
        g = _dot_t(xc, sfc)
        nx = jnp.sum(xc.astype(jnp.float32) ** 2, axis=1)
        nf = jnp.sum(sfc.astype(jnp.float32) ** 2, axis=1)
        m2 = nx[:, None] + nf[None, :] - 2.0 * g
        il = jax.lax.broadcasted_iota(jnp.int32, (B, BLK), 1)
        lidx = jnp.min(jnp.where(m2 == 0.0, il, BLK), axis=1)
        cand = jnp.where(lidx < BLK, j * BLK + lidx, N)
        midx_ref[0, :] = jnp.minimum(midx_ref[0, :], cand)

    @pl.when(j == NB - 1)
    def _gather_heads():
        ohq = (midx_ref[0, :][:, None]
               == jax.lax.broadcasted_iota(jnp.int32, (B, N), 1)
               ).astype(jnp.bfloat16)
        for (df_ref, w_ref, b_ref, u_ref, _dl,
             x_ref, oh_ref, nx_ref, uc_ref, _num) in sides:
            df = df_ref[...]
            hi = df.astype(jnp.bfloat16)
            mid = (df - hi.astype(jnp.float32)).astype(jnp.bfloat16)
            xg = _dot(ohq, hi) + _dot(ohq, mid)
            x_ref[...] = xg
            nx_ref[0, :] = jnp.sum(xg * xg, axis=1)
            u = u_ref[...]
            uc_ref[...] = _chunks8(_bits(u))
            y = _dot(xg, w_ref[...], HI) + b_ref[0, :][None, :]
            ny = jnp.sum(y * y, axis=1)
            nuf = jnp.sum(u * u, axis=1)
            dq = ny[:, None] + nuf[None, :] - 2.0 * _dot_t(y, u, HI)
            mn = jnp.min(dq, axis=1, keepdims=True)
            cb = jax.lax.broadcasted_iota(jnp.int32, (B, C), 1)
            cidx = jnp.min(jnp.where(dq == mn, cb, C), axis=1)
            oh_ref[...] = (cb == cidx[:, None]).astype(jnp.bfloat16)

    @pl.when(j >= NB)
    def _agg():
        start = pl.multiple_of((j - NB) * BLK, BLK)
        slb = slb_ref[...]
        slb_ext = jnp.concatenate(
            [slb, jnp.ones((BLK, 1), jnp.float32)], axis=1).astype(jnp.bfloat16)
        for (df_ref, _w, _b, _u, dl_ref,
             x_ref, oh_ref, nx_ref, uc_ref, num_ref) in sides:
            f = df_ref[pl.ds(start, BLK), :]
            uc = uc_ref[...]
            lc = _chunks8(_bits(dl_ref[...]))
            nl = jnp.sum(lc.astype(jnp.float32) ** 2, axis=1)
            nu = jnp.sum(uc.astype(jnp.float32) ** 2, axis=1)
            m2l = nl[:, None] + nu[None, :] - 2.0 * _dot_t(lc, uc)
            e = (m2l == 0.0).astype(jnp.bfloat16)
            mask = _dot_t(oh_ref[...], e)
            g = _dot_t(x_ref[...].astype(jnp.bfloat16),
                       f.astype(jnp.bfloat16))
            nf = jnp.sum(f * f, axis=1)
            sq = nx_ref[0, :][:, None] + nf[None, :] - 2.0 * g
            expo = (jnp.exp(-sq) * mask).astype(jnp.bfloat16)
            num_ref[...] += _dot(expo, slb_ext)

    @pl.when(j == 2 * NB - 1)
    def _fin():
        n1 = num1_ref[...]
        n2 = num2_ref[...]
        out_ref[...] = 0.5 * (n1[:, :LD] / n1[:, LD:LD + 1]
                              + n2[:, :LD] / n2[:, LD:LD + 1])


def kernel(x, star_features, star_labels, d1_features, d1_labels,
           d2_features, d2_labels, unique1, unique2, W1, b1, W2, b2):
    f32 = jnp.float32
    bf16 = jnp.bfloat16
    const2 = lambda j: (0, 0)
    matchmap = lambda j: (jnp.minimum(j, NB - 1), 0)
    aggmap = lambda j: (jnp.maximum(j - NB, 0), 0)
    s = pl.pallas_call(
        _fused_kernel,
        grid=(2 * NB,),
        in_specs=[
            pl.BlockSpec((B, D), const2),
            pl.BlockSpec((BLK, D), matchmap),
            pl.BlockSpec((N, D), const2),
            pl.BlockSpec((N, D), const2),
            pl.BlockSpec((D, LD), const2),
            pl.BlockSpec((1, LD), const2),
            pl.BlockSpec((D, LD), const2),
            pl.BlockSpec((1, LD), const2),
            pl.BlockSpec((C, LD), const2),
            pl.BlockSpec((C, LD), const2),
            pl.BlockSpec((BLK, LD), aggmap),
            pl.BlockSpec((BLK, LD), aggmap),
            pl.BlockSpec((BLK, LD), aggmap),
        ],
        out_specs=pl.BlockSpec((B, LD), const2),
        out_shape=jax.ShapeDtypeStruct((B, LD), f32),
        scratch_shapes=[
            pltpu.VMEM((B, CD), bf16),
            pltpu.VMEM((1, B), jnp.int32),
            pltpu.VMEM((B, D), f32),
            pltpu.VMEM((B, D), f32),
            pltpu.VMEM((B, C), bf16),
            pltpu.VMEM((B, C), bf16),
            pltpu.VMEM((1, B), f32),
            pltpu.VMEM((1, B), f32),
            pltpu.VMEM((C, CLD), bf16),
            pltpu.VMEM((C, CLD), bf16),
            pltpu.VMEM((B, LD + 1), f32),
            pltpu.VMEM((B, LD + 1), f32),
        ],
    )(x, star_features, d1_features, d2_features, W1, b1.reshape(1, LD),
      W2, b2.reshape(1, LD), unique1, unique2,
      d1_labels, d2_labels, star_labels)
    return s


# speedup vs baseline: 1.0671x; 1.0494x over previous
import jax
import jax.numpy as jnp
from jax.experimental import pallas as pl
from jax.experimental.pallas import tpu as pltpu

N, B, D, LD, C = 8192, 128, 128, 32, 64
BLK = 2048
NB = N // BLK
CD = D * 5
CLD = LD * 4
HI = jax.lax.Precision.HIGHEST


def _chunks7(v):
    parts = [((v >> s) & 127).astype(jnp.bfloat16) for s in (0, 7, 14, 21, 28)]
    return jnp.concatenate(parts, axis=-1)


def _chunks8(v):
    parts = [((v >> s) & 255).astype(jnp.bfloat16) for s in (0, 8, 16, 24)]
    return jnp.concatenate(parts, axis=-1)


def _bits(f):
    return jax.lax.bitcast_convert_type(f, jnp.int32)


def _dot_t(a, b, prec=None):
    return jax.lax.dot_general(a, b, (((1,), (1,)), ((), ())),
                               precision=prec, preferred_element_type=jnp.float32)


def _dot(a, b, prec=None):
    return jax.lax.dot_general(a, b, (((1,), (0,)), ((), ())),
                               precision=prec, preferred_element_type=jnp.float32)


def _fused_kernel(x_ref, sf_ref, d1a_ref, d2a_ref, w1_ref, b1_ref,
                  w2_ref, b2_ref, u1_ref, u2_ref,
                  d1f_ref, d2f_ref, d1l_ref, d2l_ref, slb_ref, out_ref,
                  xc_ref, midx_ref, mism_ref, x1_ref, x2_ref, oh1_ref, oh2_ref,
                  nx1_ref, nx2_ref, u1c_ref, u2c_ref, num1_ref, num2_ref,
                  sem_ref, gsem_ref):
    j = pl.program_id(0)
    sides = (
        (d1f_ref, w1_ref, b1_ref, u1_ref, d1l_ref,
         x1_ref, oh1_ref, nx1_ref, u1c_ref, num1_ref),
        (d2f_ref, w2_ref, b2_ref, u2_ref, d2l_ref,
         x2_ref, oh2_ref, nx2_ref, u2c_ref, num2_ref),
    )

    @pl.when(j == 0)
    def _init():
        xc_ref[...] = _chunks7(_bits(x_ref[...]))
        midx_ref[...] = jnp.full_like(midx_ref, N)
        num1_ref[...] = jnp.zeros_like(num1_ref)
        num2_ref[...] = jnp.zeros_like(num2_ref)

    @pl.when(j < NB)
    def _match():
        sfc = _chunks7(_bits(sf_ref[...]))
        xc = xc_ref[...]
        g = _dot_t(xc, sfc)
        nx = jnp.sum(xc.astype(jnp.float32) ** 2, axis=1)
        nf = jnp.sum(sfc.astype(jnp.float32) ** 2, axis=1)
        m2 = nx[:, None] + nf[None, :] - 2.0 * g
        il = jax.lax.broadcasted_iota(jnp.int32, (B, BLK), 1)
        lidx = jnp.min(jnp.where(m2 == 0.0, il, BLK), axis=1)
        cand = jnp.where(lidx < BLK, j * BLK + lidx, N)
        midx_ref[0, :] = jnp.minimum(midx_ref[0, :], cand)

    @pl.when(j == NB - 1)
    def _gather():
        pltpu.make_async_copy(midx_ref, mism_ref, sem_ref).start()
        pltpu.make_async_copy(midx_ref, mism_ref, sem_ref).wait()
        for i in range(B):
            idx = mism_ref[0, i]
            pltpu.make_async_copy(d1a_ref.at[pl.ds(idx, 1), :],
                                  x1_ref.at[pl.ds(i, 1), :], gsem_ref).start()
            pltpu.make_async_copy(d2a_ref.at[pl.ds(idx, 1), :],
                                  x2_ref.at[pl.ds(i, 1), :], gsem_ref).start()

    @pl.when(j == NB)
    def _heads():
        for i in range(B):
            pltpu.make_async_copy(d1a_ref.at[pl.ds(0, 1), :],
                                  x1_ref.at[pl.ds(i, 1), :], gsem_ref).wait()
            pltpu.make_async_copy(d2a_ref.at[pl.ds(0, 1), :],
                                  x2_ref.at[pl.ds(i, 1), :], gsem_ref).wait()
        for (_df, w_ref, b_ref, u_ref, _dl,
             x_ref, oh_ref, nx_ref, uc_ref, _num) in sides:
            xg = x_ref[...]
            nx_ref[0, :] = jnp.sum(xg * xg, axis=1)
            u = u_ref[...]
            uc_ref[...] = _chunks8(_bits(u))
            y = _dot(xg, w_ref[...], HI) + b_ref[0, :][None, :]
            ny = jnp.sum(y * y, axis=1)
            nuf = jnp.sum(u * u, axis=1)
            dq = ny[:, None] + nuf[None, :] - 2.0 * _dot_t(y, u, HI)
            mn = jnp.min(dq, axis=1, keepdims=True)
            cb = jax.lax.broadcasted_iota(jnp.int32, (B, C), 1)
            cidx = jnp.min(jnp.where(dq == mn, cb, C), axis=1)
            oh_ref[...] = (cb == cidx[:, None]).astype(jnp.bfloat16)

    @pl.when(j >= NB)
    def _agg():
        slb = slb_ref[...]
        slb_ext = jnp.concatenate(
            [slb, jnp.ones((BLK, 1), jnp.float32)], axis=1).astype(jnp.bfloat16)
        for (df_ref, _w, _b, _u, dl_ref,
             x_ref, oh_ref, nx_ref, uc_ref, num_ref) in sides:
            f = df_ref[...]
            uc = uc_ref[...]
            lc = _chunks8(_bits(dl_ref[...]))
            nl = jnp.sum(lc.astype(jnp.float32) ** 2, axis=1)
            nu = jnp.sum(uc.astype(jnp.float32) ** 2, axis=1)
            m2l = nl[:, None] + nu[None, :] - 2.0 * _dot_t(lc, uc)
            e = (m2l == 0.0).astype(jnp.bfloat16)
            mask = _dot_t(oh_ref[...], e)
            g = _dot_t(x_ref[...].astype(jnp.bfloat16),
                       f.astype(jnp.bfloat16))
            nf = jnp.sum(f * f, axis=1)
            sq = nx_ref[0, :][:, None] + nf[None, :] - 2.0 * g
            expo = (jnp.exp(-sq) * mask).astype(jnp.bfloat16)
            num_ref[...] += _dot(expo, slb_ext)

    @pl.when(j == 2 * NB - 1)
    def _fin():
        n1 = num1_ref[...]
        n2 = num2_ref[...]
        out_ref[...] = 0.5 * (n1[:, :LD] / n1[:, LD:LD + 1]
                              + n2[:, :LD] / n2[:, LD:LD + 1])


def kernel(x, star_features, star_labels, d1_features, d1_labels,
           d2_features, d2_labels, unique1, unique2, W1, b1, W2, b2):
    f32 = jnp.float32
    bf16 = jnp.bfloat16
    const2 = lambda j: (0, 0)
    matchmap = lambda j: (jnp.minimum(j, NB - 1), 0)
    aggmap = lambda j: (jnp.maximum(j - NB, 0), 0)
    s = pl.pallas_call(
        _fused_kernel,
        grid=(2 * NB,),
        in_specs=[
            pl.BlockSpec((B, D), const2),
            pl.BlockSpec((BLK, D), matchmap),
            pl.BlockSpec(memory_space=pltpu.MemorySpace.HBM),
            pl.BlockSpec(memory_space=pltpu.MemorySpace.HBM),
            pl.BlockSpec((D, LD), const2),
            pl.BlockSpec((1, LD), const2),
            pl.BlockSpec((D, LD), const2),
            pl.BlockSpec((1, LD), const2),
            pl.BlockSpec((C, LD), const2),
            pl.BlockSpec((C, LD), const2),
            pl.BlockSpec((BLK, D), aggmap),
            pl.BlockSpec((BLK, D), aggmap),
            pl.BlockSpec((BLK, LD), aggmap),
            pl.BlockSpec((BLK, LD), aggmap),
            pl.BlockSpec((BLK, LD), aggmap),
        ],
        out_specs=pl.BlockSpec((B, LD), const2),
        out_shape=jax.ShapeDtypeStruct((B, LD), f32),
        scratch_shapes=[
            pltpu.VMEM((B, CD), bf16),
            pltpu.VMEM((1, B), jnp.int32),
            pltpu.SMEM((1, B), jnp.int32),
            pltpu.VMEM((B, D), f32),
            pltpu.VMEM((B, D), f32),
            pltpu.VMEM((B, C), bf16),
            pltpu.VMEM((B, C), bf16),
            pltpu.VMEM((1, B), f32),
            pltpu.VMEM((1, B), f32),
            pltpu.VMEM((C, CLD), bf16),
            pltpu.VMEM((C, CLD), bf16),
            pltpu.VMEM((B, LD + 1), f32),
            pltpu.VMEM((B, LD + 1), f32),
            pltpu.SemaphoreType.DMA,
            pltpu.SemaphoreType.DMA,
        ],
    )(x, star_features, d1_features, d2_features, W1, b1.reshape(1, LD),
      W2, b2.reshape(1, LD), unique1, unique2,
      d1_features, d2_features, d1_labels, d2_labels, star_labels)
    return s


# speedup vs baseline: 1.0723x; 1.0049x over previous
import jax
import jax.numpy as jnp
from jax.experimental import pallas as pl
from jax.experimental.pallas import tpu as pltpu

N, B, D, LD, C = 8192, 128, 128, 32, 64
BLK = 2048
NB = N // BLK
CD = D * 5
CLD = LD * 4
HI = jax.lax.Precision.HIGHEST


def _chunks7(v):
    parts = [((v >> s) & 127).astype(jnp.bfloat16) for s in (0, 7, 14, 21, 28)]
    return jnp.concatenate(parts, axis=-1)


def _chunks8(v):
    parts = [((v >> s) & 255).astype(jnp.bfloat16) for s in (0, 8, 16, 24)]
    return jnp.concatenate(parts, axis=-1)


def _bits(f):
    return jax.lax.bitcast_convert_type(f, jnp.int32)


def _dot_t(a, b, prec=None):
    return jax.lax.dot_general(a, b, (((1,), (1,)), ((), ())),
                               precision=prec, preferred_element_type=jnp.float32)


def _dot(a, b, prec=None):
    return jax.lax.dot_general(a, b, (((1,), (0,)), ((), ())),
                               precision=prec, preferred_element_type=jnp.float32)


def _match_kernel(x_ref, sf_ref, midx_ref, xc_ref):
    j = pl.program_id(0)

    @pl.when(j == 0)
    def _init():
        xc_ref[...] = _chunks7(_bits(x_ref[...]))
        midx_ref[...] = jnp.full_like(midx_ref, N)

    sfc = _chunks7(_bits(sf_ref[...]))
    xc = xc_ref[...]
    g = _dot_t(xc, sfc)
    nx = jnp.sum(xc.astype(jnp.float32) ** 2, axis=1)
    nf = jnp.sum(sfc.astype(jnp.float32) ** 2, axis=1)
    m2 = nx[:, None] + nf[None, :] - 2.0 * g
    il = jax.lax.broadcasted_iota(jnp.int32, (B, BLK), 1)
    lidx = jnp.min(jnp.where(m2 == 0.0, il, BLK), axis=1)
    cand = jnp.where(lidx < BLK, j * BLK + lidx, N)
    midx_ref[0, :] = jnp.minimum(midx_ref[0, :], cand)


def _gather_heads_kernel(midx_ref, d1a_ref, d2a_ref, w1_ref, b1_ref,
                         w2_ref, b2_ref, u1_ref, u2_ref,
                         x1_ref, x2_ref, oh1_ref, oh2_ref,
                         nx1_ref, nx2_ref, u1c_ref, u2c_ref,
                         mism_ref, sem_ref, gsem_ref):
    pltpu.make_async_copy(midx_ref, mism_ref, sem_ref).start()
    pltpu.make_async_copy(midx_ref, mism_ref, sem_ref).wait()
    for i in range(B):
        idx = mism_ref[0, i]
        pltpu.make_async_copy(d1a_ref.at[pl.ds(idx, 1), :],
                              x1_ref.at[pl.ds(i, 1), :], gsem_ref).start()
        pltpu.make_async_copy(d2a_ref.at[pl.ds(idx, 1), :],
                              x2_ref.at[pl.ds(i, 1), :], gsem_ref).start()
    for i in range(B):
        pltpu.make_async_copy(d1a_ref.at[pl.ds(0, 1), :],
                              x1_ref.at[pl.ds(i, 1), :], gsem_ref).wait()
        pltpu.make_async_copy(d2a_ref.at[pl.ds(0, 1), :],
                              x2_ref.at[pl.ds(i, 1), :], gsem_ref).wait()
    sides = (
        (w1_ref, b1_ref, u1_ref, x1_ref, oh1_ref, nx1_ref, u1c_ref),
        (w2_ref, b2_ref, u2_ref, x2_ref, oh2_ref, nx2_ref, u2c_ref),
    )
    for (w_ref, b_ref, u_ref, x_ref, oh_ref, nx_ref, uc_ref) in sides:
        xg = x_ref[...]
        nx_ref[0, :] = jnp.sum(xg * xg, axis=1)
        u = u_ref[...]
        uc_ref[...] = _chunks8(_bits(u))
        y = _dot(xg, w_ref[...], HI) + b_ref[0, :][None, :]
        ny = jnp.sum(y * y, axis=1)
        nuf = jnp.sum(u * u, axis=1)
        dq = ny[:, None] + nuf[None, :] - 2.0 * _dot_t(y, u, HI)
        mn = jnp.min(dq, axis=1, keepdims=True)
        cb = jax.lax.broadcasted_iota(jnp.int32, (B, C), 1)
        cidx = jnp.min(jnp.where(dq == mn, cb, C), axis=1)
        oh_ref[...] = (cb == cidx[:, None]).astype(jnp.bfloat16)


def _agg_kernel(x1_ref, x2_ref, oh1_ref, oh2_ref, nx1_ref, nx2_ref,
                u1c_ref, u2c_ref,
                d1f_ref, d1l_ref, d2f_ref, d2l_ref, slb_ref, out_ref,
                num1_ref, num2_ref):
    j = pl.program_id(0)

    @pl.when(j == 0)
    def _init():
        num1_ref[...] = jnp.zeros_like(num1_ref)
        num2_ref[...] = jnp.zeros_like(num2_ref)

    slb = slb_ref[...]
    slb_ext = jnp.concatenate(
        [slb, jnp.ones((BLK, 1), jnp.float32)], axis=1).astype(jnp.bfloat16)
    sides = (
        (x1_ref, oh1_ref, nx1_ref, u1c_ref, d1f_ref, d1l_ref, num1_ref),
        (x2_ref, oh2_ref, nx2_ref, u2c_ref, d2f_ref, d2l_ref, num2_ref),
    )
    for (x_ref, oh_ref, nx_ref, uc_ref, df_ref, dl_ref, num_ref) in sides:
        f = df_ref[...]
        uc = uc_ref[...]
        lc = _chunks8(_bits(dl_ref[...]))
        nl = jnp.sum(lc.astype(jnp.float32) ** 2, axis=1)
        nu = jnp.sum(uc.astype(jnp.float32) ** 2, axis=1)
        m2l = nl[:, None] + nu[None, :] - 2.0 * _dot_t(lc, uc)
        e = (m2l == 0.0).astype(jnp.bfloat16)
        mask = _dot_t(oh_ref[...], e)
        g = _dot_t(x_ref[...].astype(jnp.bfloat16),
                   f.astype(jnp.bfloat16))
        nf = jnp.sum(f * f, axis=1)
        sq = nx_ref[0, :][:, None] + nf[None, :] - 2.0 * g
        expo = (jnp.exp(-sq) * mask).astype(jnp.bfloat16)
        num_ref[...] += _dot(expo, slb_ext)

    @pl.when(j == NB - 1)
    def _fin():
        n1 = num1_ref[...]
        n2 = num2_ref[...]
        out_ref[...] = 0.5 * (n1[:, :LD] / n1[:, LD:LD + 1]
                              + n2[:, :LD] / n2[:, LD:LD + 1])


def kernel(x, star_features, star_labels, d1_features, d1_labels,
           d2_features, d2_labels, unique1, unique2, W1, b1, W2, b2):
    f32 = jnp.float32
    bf16 = jnp.bfloat16
    const2 = lambda j: (0, 0)
    midx = pl.pallas_call(
        _match_kernel,
        grid=(NB,),
        in_specs=[
            pl.BlockSpec((B, D), const2),
            pl.BlockSpec((BLK, D), lambda j: (j, 0)),
        ],
        out_specs=pl.BlockSpec((1, B), const2),
        out_shape=jax.ShapeDtypeStruct((1, B), jnp.int32),
        scratch_shapes=[
            pltpu.VMEM((B, CD), bf16),
        ],
    )(x, star_features)

    x1, x2, oh1, oh2, nx1, nx2, u1c, u2c = pl.pallas_call(
        _gather_heads_kernel,
        in_specs=[
            pl.BlockSpec((1, B), None),
            pl.BlockSpec(memory_space=pltpu.MemorySpace.HBM),
            pl.BlockSpec(memory_space=pltpu.MemorySpace.HBM),
            pl.BlockSpec((D, LD), None),
            pl.BlockSpec((1, LD), None),
            pl.BlockSpec((D, LD), None),
            pl.BlockSpec((1, LD), None),
            pl.BlockSpec((C, LD), None),
            pl.BlockSpec((C, LD), None),
        ],
        out_shape=[
            jax.ShapeDtypeStruct((B, D), f32),
            jax.ShapeDtypeStruct((B, D), f32),
            jax.ShapeDtypeStruct((B, C), bf16),
            jax.ShapeDtypeStruct((B, C), bf16),
            jax.ShapeDtypeStruct((1, B), f32),
            jax.ShapeDtypeStruct((1, B), f32),
            jax.ShapeDtypeStruct((C, CLD), bf16),
            jax.ShapeDtypeStruct((C, CLD), bf16),
        ],
        scratch_shapes=[
            pltpu.SMEM((1, B), jnp.int32),
            pltpu.SemaphoreType.DMA,
            pltpu.SemaphoreType.DMA,
        ],
    )(midx, d1_features, d2_features, W1, b1.reshape(1, LD),
      W2, b2.reshape(1, LD), unique1, unique2)

    s = pl.pallas_call(
        _agg_kernel,
        grid=(NB,),
        in_specs=[
            pl.BlockSpec((B, D), const2),
            pl.BlockSpec((B, D), const2),
            pl.BlockSpec((B, C), const2),
            pl.BlockSpec((B, C), const2),
            pl.BlockSpec((1, B), const2),
            pl.BlockSpec((1, B), const2),
            pl.BlockSpec((C, CLD), const2),
            pl.BlockSpec((C, CLD), const2),
            pl.BlockSpec((BLK, D), lambda j: (j, 0)),
            pl.BlockSpec((BLK, LD), lambda j: (j, 0)),
            pl.BlockSpec((BLK, D), lambda j: (j, 0)),
            pl.BlockSpec((BLK, LD), lambda j: (j, 0)),
            pl.BlockSpec((BLK, LD), lambda j: (j, 0)),
        ],
        out_specs=pl.BlockSpec((B, LD), const2),
        out_shape=jax.ShapeDtypeStruct((B, LD), f32),
        scratch_shapes=[
            pltpu.VMEM((B, LD + 1), f32),
            pltpu.VMEM((B, LD + 1), f32),
        ],
    )(x1, x2, oh1, oh2, nx1, nx2, u1c, u2c,
      d1_features, d1_labels, d2_features, d2_labels, star_labels)
    return s
